# baseline (speedup 1.0000x reference)
import functools

import jax
import jax.numpy as jnp
from jax import lax
from jax.experimental import pallas as pl
from jax.experimental.pallas import tpu as pltpu
from jax.experimental.pallas import tpu_sc as plsc

N = 10000
E = 320000
G = 64
H = 128

NC = 2
NS = 16
NW = NC * NS
CHUNK = 128
EPW = -(-(E // NW) // (CHUNK * 4)) * (CHUNK * 4)
NCHUNK = EPW // CHUNK
EPAD = EPW * NW
ACC_ROWS = EPW
RPT = ACC_ROWS // NS

BLK = 2000
NBLK = N // BLK

_sc_mesh = functools.partial(
    plsc.VectorSubcoreMesh, core_axis_name="c", subcore_axis_name="s",
    num_cores=NC, num_subcores=NS)


def _sc_degree_body(dst_hbm, ones_hbm, zeros_hbm, out_hbm, acc, didx, ones_v):
  c = lax.axis_index("c")
  s = lax.axis_index("s")
  wid = s * NC + c
  pltpu.sync_copy(ones_hbm, ones_v)
  pltpu.sync_copy(zeros_hbm, acc.at[pl.ds(s * RPT, RPT)])
  plsc.subcore_barrier()
  for p in range(PH):
    pltpu.sync_copy(dst_hbm.at[wid, p], didx)

    def body(j, _):
      pltpu.sync_copy(ones_v, acc.at[didx.at[j]], add=True)
      return ()

    lax.fori_loop(0, CPP, body, ())
  plsc.subcore_barrier()
  pltpu.sync_copy(acc.at[pl.ds(s * RPT, RPT)],
                  out_hbm.at[c, pl.ds(s * RPT, RPT)])


@functools.cache
def _make_sc_degree():
  return pl.kernel(
      _sc_degree_body,
      out_type=jax.ShapeDtypeStruct((NC, ACC_ROWS, H), jnp.float32),
      mesh=_sc_mesh(),
      scratch_types=[
          pltpu.VMEM_SHARED((ACC_ROWS, H), jnp.float32),
          pltpu.VMEM((CPP, CHUNK), jnp.int32),
          pltpu.VMEM((CHUNK, H), jnp.float32),
      ],
  )


def _sc_degree(*args):
  return _make_sc_degree()(*args)


NRING = 2
PH = 2
CPP = NCHUNK // PH


def _sc_aggregate_body(g_hbm, src_hbm, dst_hbm, zeros_hbm, out_hbm,
                       acc, sidx, didx, rows):
  c = lax.axis_index("c")
  s = lax.axis_index("s")
  wid = s * NC + c
  pltpu.sync_copy(src_hbm.at[wid], sidx)
  pltpu.sync_copy(dst_hbm.at[wid], didx)
  pltpu.sync_copy(zeros_hbm, acc.at[pl.ds(s * RPT, RPT)])
  plsc.subcore_barrier()
  for p in range(PH):

    def body(j, _):
      pltpu.sync_copy(g_hbm.at[sidx.at[p, j]], rows)
      pltpu.sync_copy(rows, acc.at[didx.at[p, j]], add=True)
      return ()

    lax.fori_loop(0, CPP, body, ())
  plsc.subcore_barrier()
  pltpu.sync_copy(acc.at[pl.ds(s * RPT, RPT)],
                  out_hbm.at[c, pl.ds(s * RPT, RPT)])


@functools.cache
def _make_sc_aggregate():
  return pl.kernel(
      _sc_aggregate_body,
      out_type=jax.ShapeDtypeStruct((NC, ACC_ROWS, H), jnp.float32),
      mesh=_sc_mesh(),
      scratch_types=[
          pltpu.VMEM_SHARED((ACC_ROWS, H), jnp.float32),
          pltpu.VMEM((PH, CPP, CHUNK), jnp.int32),
          pltpu.VMEM((PH, CPP, CHUNK), jnp.int32),
          pltpu.VMEM((CHUNK, H), jnp.float32),
      ],
  )


def _sc_aggregate(*args):
  return _make_sc_aggregate()(*args)


def _tc_first_body(x_ref, w_ref, dp0_ref, dp1_ref, g_ref, dis_ref):
  deg = dp0_ref[:, 0:1] + dp1_ref[:, 0:1] + 1.0
  dis = lax.rsqrt(deg)
  a = jnp.dot(x_ref[...], w_ref[...], preferred_element_type=jnp.float32)
  g_ref[...] = a * dis
  dis_ref[...] = jnp.broadcast_to(dis, (BLK, H))


def _tc_first(x, w, dp0, dp1):
  return pl.pallas_call(
      _tc_first_body,
      grid=(NBLK,),
      in_specs=[
          pl.BlockSpec((BLK, H), lambda i: (i, 0)),
          pl.BlockSpec((H, H), lambda i: (0, 0)),
          pl.BlockSpec((BLK, H), lambda i: (i, 0)),
          pl.BlockSpec((BLK, H), lambda i: (i, 0)),
      ],
      out_specs=[
          pl.BlockSpec((BLK, H), lambda i: (i, 0)),
          pl.BlockSpec((BLK, H), lambda i: (i, 0)),
      ],
      out_shape=[
          jax.ShapeDtypeStruct((N, H), jnp.float32),
          jax.ShapeDtypeStruct((N, H), jnp.float32),
      ],
  )(x, w, dp0, dp1)


def _tc_combine_body(p0_ref, p1_ref, g_ref, dis_ref, b_ref, w_ref, out_ref):
  h = (p0_ref[...] + p1_ref[...] + g_ref[...]) * dis_ref[...] + b_ref[...]
  h = jnp.maximum(h, 0.0)
  a = jnp.dot(h, w_ref[...], preferred_element_type=jnp.float32)
  out_ref[...] = a * dis_ref[...]


def _tc_combine(p0, p1, g, dis, b, w):
  return pl.pallas_call(
      _tc_combine_body,
      grid=(NBLK,),
      in_specs=[
          pl.BlockSpec((BLK, H), lambda i: (i, 0)),
          pl.BlockSpec((BLK, H), lambda i: (i, 0)),
          pl.BlockSpec((BLK, H), lambda i: (i, 0)),
          pl.BlockSpec((BLK, H), lambda i: (i, 0)),
          pl.BlockSpec((1, H), lambda i: (0, 0)),
          pl.BlockSpec((H, H), lambda i: (0, 0)),
      ],
      out_specs=pl.BlockSpec((BLK, H), lambda i: (i, 0)),
      out_shape=jax.ShapeDtypeStruct((N, H), jnp.float32),
  )(p0, p1, g, dis, b, w)


def _tc_pool_mlp_body(p0_ref, p1_ref, g_ref, dis_ref, b_ref, batch_ref,
                      gf_ref, m1a_ref, m1g_ref, b1_ref, m2_ref, b2_ref,
                      m3_ref, b3_ref, out_ref, sums, cnt):
  i = pl.program_id(0)

  @pl.when(i == 0)
  def _():
    sums[...] = jnp.zeros((G, H), jnp.float32)
    cnt[...] = jnp.zeros((G, H), jnp.float32)

  h = (p0_ref[...] + p1_ref[...] + g_ref[...]) * dis_ref[...] + b_ref[...]
  h = jnp.maximum(h, 0.0)
  ids = lax.broadcasted_iota(jnp.int32, (BLK, G), 1)
  oh = (batch_ref[...] == ids).astype(jnp.float32)
  dn = (((0,), (0,)), ((), ()))
  sums[...] += lax.dot_general(oh, h, dn, preferred_element_type=jnp.float32)
  cnt[...] += lax.dot_general(oh, jnp.ones((BLK, H), jnp.float32), dn,
                              preferred_element_type=jnp.float32)

  @pl.when(i == NBLK - 1)
  def _():
    pooled = sums[...] / jnp.maximum(cnt[...], 1.0)
    z = jnp.dot(pooled, m1a_ref[...], preferred_element_type=jnp.float32)
    z += jnp.dot(gf_ref[...], m1g_ref[...], preferred_element_type=jnp.float32)
    z = jnp.maximum(z + b1_ref[...], 0.0)
    z = jnp.dot(z, m2_ref[...], preferred_element_type=jnp.float32)
    z = jnp.maximum(z + b2_ref[...], 0.0)
    z = jnp.dot(z, m3_ref[...], preferred_element_type=jnp.float32)
    out_ref[...] = z + b3_ref[...]


def _tc_pool_mlp(p0, p1, g, dis, b, batch2d, gfp, m1a, m1g, b1p, m2p, b2p,
                 m3p, b3p):
  full = lambda r, c: pl.BlockSpec((r, c), lambda i: (0, 0))
  return pl.pallas_call(
      _tc_pool_mlp_body,
      grid=(NBLK,),
      in_specs=[
          pl.BlockSpec((BLK, H), lambda i: (i, 0)),
          pl.BlockSpec((BLK, H), lambda i: (i, 0)),
          pl.BlockSpec((BLK, H), lambda i: (i, 0)),
          pl.BlockSpec((BLK, H), lambda i: (i, 0)),
          full(1, H),
          pl.BlockSpec((BLK, 1), lambda i: (i, 0)),
          full(G, H), full(H, H), full(H, H), full(1, H),
          full(H, H), full(1, H), full(H, H), full(1, H),
      ],
      out_specs=pl.BlockSpec((G, H), lambda i: (0, 0)),
      out_shape=jax.ShapeDtypeStruct((G, H), jnp.float32),
      scratch_shapes=[
          pltpu.VMEM((G, H), jnp.float32),
          pltpu.VMEM((G, H), jnp.float32),
      ],
  )(p0, p1, g, dis, b, batch2d, gfp, m1a, m1g, b1p, m2p, b2p, m3p, b3p)


def _pad2(a, rows, cols):
  return jnp.zeros((rows, cols), jnp.float32).at[:a.shape[0], :a.shape[1]].set(a)


def kernel(x, edge_index, batch, global_features, W1, b1, W2, b2, W3, b3,
           m1w, m1b, m2w, m2b, m3w, m3b):
  src = edge_index[0].astype(jnp.int32)
  dst = edge_index[1].astype(jnp.int32)
  src_p = jnp.concatenate(
      [src, jnp.zeros((EPAD - E,), jnp.int32)]).reshape(NW, PH, CPP, CHUNK)
  dst_p = jnp.concatenate(
      [dst, jnp.full((EPAD - E,), N, jnp.int32)]).reshape(NW, PH, CPP, CHUNK)

  zerosH = jnp.zeros((RPT, H), jnp.float32)
  onesH = jnp.ones((CHUNK, H), jnp.float32)

  degp = _sc_degree(dst_p, onesH, zerosH)
  dp0 = degp[0]
  dp1 = degp[1]

  g1, dis = _tc_first(x, W1, dp0, dp1)

  b1r = b1.reshape(1, H)
  b2r = b2.reshape(1, H)
  b3r = b3.reshape(1, H)

  a1 = _sc_aggregate(g1, src_p, dst_p, zerosH)
  g2 = _tc_combine(a1[0, :N], a1[1, :N], g1, dis, b1r, W2)
  a2 = _sc_aggregate(g2, src_p, dst_p, zerosH)
  g3 = _tc_combine(a2[0, :N], a2[1, :N], g2, dis, b2r, W3)
  a3 = _sc_aggregate(g3, src_p, dst_p, zerosH)

  batch2d = batch.astype(jnp.int32).reshape(N, 1)
  gfp = _pad2(global_features, G, H)
  m1a = _pad2(m1w[:H], H, H)
  m1g = _pad2(m1w[H:], H, H)
  b1p = _pad2(m1b.reshape(1, -1), 1, H)
  m2p = _pad2(m2w, H, H)
  b2p = _pad2(m2b.reshape(1, -1), 1, H)
  m3p = _pad2(m3w, H, H)
  b3p = _pad2(m3b.reshape(1, -1), 1, H)

  z = _tc_pool_mlp(a3[0, :N], a3[1, :N], g3, dis, b3r, batch2d, gfp,
                   m1a, m1g, b1p, m2p, b2p, m3p, b3p)
  return z[:, :1]


# speedup vs baseline: 1.0007x; 1.0007x over previous
import functools

import jax
import jax.numpy as jnp
from jax import lax
from jax.experimental import pallas as pl
from jax.experimental.pallas import tpu as pltpu
from jax.experimental.pallas import tpu_sc as plsc

N = 10000
E = 320000
G = 64
H = 128

NC = 2
NS = 16
NW = NC * NS
CHUNK = 128
EPW = -(-(E // NW) // (CHUNK * 4)) * (CHUNK * 4)
NCHUNK = EPW // CHUNK
EPAD = EPW * NW
ACC_ROWS = EPW
RPT = ACC_ROWS // NS

BLK = 2000
NBLK = N // BLK

_sc_mesh = functools.partial(
    plsc.VectorSubcoreMesh, core_axis_name="c", subcore_axis_name="s",
    num_cores=NC, num_subcores=NS)


def _sc_degree_body(dst_hbm, ones_hbm, zeros_hbm, out_hbm, acc, didx, ones_v):
  c = lax.axis_index("c")
  s = lax.axis_index("s")
  wid = s * NC + c
  pltpu.sync_copy(dst_hbm.at[wid], didx)
  pltpu.sync_copy(ones_hbm, ones_v)
  pltpu.sync_copy(zeros_hbm, acc.at[pl.ds(s * RPT, RPT)])
  plsc.subcore_barrier()

  def body(j, _):
    pltpu.sync_copy(ones_v, acc.at[didx.at[j]], add=True)
    return ()

  lax.fori_loop(0, NCHUNK, body, ())
  plsc.subcore_barrier()
  pltpu.sync_copy(acc.at[pl.ds(s * RPT, RPT)],
                  out_hbm.at[c, pl.ds(s * RPT, RPT)])


@functools.cache
def _make_sc_degree():
  return pl.kernel(
      _sc_degree_body,
      out_type=jax.ShapeDtypeStruct((NC, ACC_ROWS, H), jnp.float32),
      mesh=_sc_mesh(),
      scratch_types=[
          pltpu.VMEM_SHARED((ACC_ROWS, H), jnp.float32),
          pltpu.VMEM((NCHUNK, CHUNK), jnp.int32),
          pltpu.VMEM((CHUNK, H), jnp.float32),
      ],
  )


def _sc_degree(*args):
  return _make_sc_degree()(*args)


NRING = 2
PH = 2
CPP = NCHUNK // PH


def _sc_aggregate_body(g_hbm, src_hbm, dst_hbm, zeros_hbm, out_hbm,
                       acc, sidx, didx, rows):
  c = lax.axis_index("c")
  s = lax.axis_index("s")
  wid = s * NC + c
  pltpu.sync_copy(src_hbm.at[wid], sidx)
  pltpu.sync_copy(dst_hbm.at[wid], didx)
  pltpu.sync_copy(zeros_hbm, acc.at[pl.ds(s * RPT, RPT)])
  plsc.subcore_barrier()

  def body(j, _):
    pltpu.sync_copy(g_hbm.at[sidx.at[j]], rows)
    pltpu.sync_copy(rows, acc.at[didx.at[j]], add=True)
    return ()

  lax.fori_loop(0, NCHUNK, body, ())
  plsc.subcore_barrier()
  pltpu.sync_copy(acc.at[pl.ds(s * RPT, RPT)],
                  out_hbm.at[c, pl.ds(s * RPT, RPT)])


@functools.cache
def _make_sc_aggregate():
  return pl.kernel(
      _sc_aggregate_body,
      out_type=jax.ShapeDtypeStruct((NC, ACC_ROWS, H), jnp.float32),
      mesh=_sc_mesh(),
      scratch_types=[
          pltpu.VMEM_SHARED((ACC_ROWS, H), jnp.float32),
          pltpu.VMEM((NCHUNK, CHUNK), jnp.int32),
          pltpu.VMEM((NCHUNK, CHUNK), jnp.int32),
          pltpu.VMEM((CHUNK, H), jnp.float32),
      ],
  )


def _sc_aggregate(*args):
  return _make_sc_aggregate()(*args)


def _tc_first_body(x_ref, w_ref, dp0_ref, dp1_ref, g_ref, dis_ref):
  deg = dp0_ref[:, 0:1] + dp1_ref[:, 0:1] + 1.0
  dis = lax.rsqrt(deg)
  a = jnp.dot(x_ref[...], w_ref[...], preferred_element_type=jnp.float32)
  g_ref[...] = a * dis
  dis_ref[...] = jnp.broadcast_to(dis, (BLK, H))


def _tc_first(x, w, dp0, dp1):
  return pl.pallas_call(
      _tc_first_body,
      grid=(NBLK,),
      in_specs=[
          pl.BlockSpec((BLK, H), lambda i: (i, 0)),
          pl.BlockSpec((H, H), lambda i: (0, 0)),
          pl.BlockSpec((BLK, H), lambda i: (i, 0)),
          pl.BlockSpec((BLK, H), lambda i: (i, 0)),
      ],
      out_specs=[
          pl.BlockSpec((BLK, H), lambda i: (i, 0)),
          pl.BlockSpec((BLK, H), lambda i: (i, 0)),
      ],
      out_shape=[
          jax.ShapeDtypeStruct((N, H), jnp.float32),
          jax.ShapeDtypeStruct((N, H), jnp.float32),
      ],
  )(x, w, dp0, dp1)


def _tc_combine_body(p0_ref, p1_ref, g_ref, dis_ref, b_ref, w_ref, out_ref):
  h = (p0_ref[...] + p1_ref[...] + g_ref[...]) * dis_ref[...] + b_ref[...]
  h = jnp.maximum(h, 0.0)
  a = jnp.dot(h, w_ref[...], preferred_element_type=jnp.float32)
  out_ref[...] = a * dis_ref[...]


def _tc_combine(p0, p1, g, dis, b, w):
  return pl.pallas_call(
      _tc_combine_body,
      grid=(NBLK,),
      in_specs=[
          pl.BlockSpec((BLK, H), lambda i: (i, 0)),
          pl.BlockSpec((BLK, H), lambda i: (i, 0)),
          pl.BlockSpec((BLK, H), lambda i: (i, 0)),
          pl.BlockSpec((BLK, H), lambda i: (i, 0)),
          pl.BlockSpec((1, H), lambda i: (0, 0)),
          pl.BlockSpec((H, H), lambda i: (0, 0)),
      ],
      out_specs=pl.BlockSpec((BLK, H), lambda i: (i, 0)),
      out_shape=jax.ShapeDtypeStruct((N, H), jnp.float32),
  )(p0, p1, g, dis, b, w)


def _tc_pool_mlp_body(p0_ref, p1_ref, g_ref, dis_ref, b_ref, batch_ref,
                      gf_ref, m1a_ref, m1g_ref, b1_ref, m2_ref, b2_ref,
                      m3_ref, b3_ref, out_ref, sums, cnt):
  i = pl.program_id(0)

  @pl.when(i == 0)
  def _():
    sums[...] = jnp.zeros((G, H), jnp.float32)
    cnt[...] = jnp.zeros((G, H), jnp.float32)

  h = (p0_ref[...] + p1_ref[...] + g_ref[...]) * dis_ref[...] + b_ref[...]
  h = jnp.maximum(h, 0.0)
  ids = lax.broadcasted_iota(jnp.int32, (BLK, G), 1)
  oh = (batch_ref[...] == ids).astype(jnp.float32)
  dn = (((0,), (0,)), ((), ()))
  sums[...] += lax.dot_general(oh, h, dn, preferred_element_type=jnp.float32)
  cnt[...] += lax.dot_general(oh, jnp.ones((BLK, H), jnp.float32), dn,
                              preferred_element_type=jnp.float32)

  @pl.when(i == NBLK - 1)
  def _():
    pooled = sums[...] / jnp.maximum(cnt[...], 1.0)
    z = jnp.dot(pooled, m1a_ref[...], preferred_element_type=jnp.float32)
    z += jnp.dot(gf_ref[...], m1g_ref[...], preferred_element_type=jnp.float32)
    z = jnp.maximum(z + b1_ref[...], 0.0)
    z = jnp.dot(z, m2_ref[...], preferred_element_type=jnp.float32)
    z = jnp.maximum(z + b2_ref[...], 0.0)
    z = jnp.dot(z, m3_ref[...], preferred_element_type=jnp.float32)
    out_ref[...] = z + b3_ref[...]


def _tc_pool_mlp(p0, p1, g, dis, b, batch2d, gfp, m1a, m1g, b1p, m2p, b2p,
                 m3p, b3p):
  full = lambda r, c: pl.BlockSpec((r, c), lambda i: (0, 0))
  return pl.pallas_call(
      _tc_pool_mlp_body,
      grid=(NBLK,),
      in_specs=[
          pl.BlockSpec((BLK, H), lambda i: (i, 0)),
          pl.BlockSpec((BLK, H), lambda i: (i, 0)),
          pl.BlockSpec((BLK, H), lambda i: (i, 0)),
          pl.BlockSpec((BLK, H), lambda i: (i, 0)),
          full(1, H),
          pl.BlockSpec((BLK, 1), lambda i: (i, 0)),
          full(G, H), full(H, H), full(H, H), full(1, H),
          full(H, H), full(1, H), full(H, H), full(1, H),
      ],
      out_specs=pl.BlockSpec((G, H), lambda i: (0, 0)),
      out_shape=jax.ShapeDtypeStruct((G, H), jnp.float32),
      scratch_shapes=[
          pltpu.VMEM((G, H), jnp.float32),
          pltpu.VMEM((G, H), jnp.float32),
      ],
  )(p0, p1, g, dis, b, batch2d, gfp, m1a, m1g, b1p, m2p, b2p, m3p, b3p)


def _pad2(a, rows, cols):
  return jnp.zeros((rows, cols), jnp.float32).at[:a.shape[0], :a.shape[1]].set(a)


def kernel(x, edge_index, batch, global_features, W1, b1, W2, b2, W3, b3,
           m1w, m1b, m2w, m2b, m3w, m3b):
  src = edge_index[0].astype(jnp.int32)
  dst = edge_index[1].astype(jnp.int32)
  src_p = jnp.concatenate(
      [src, jnp.zeros((EPAD - E,), jnp.int32)]).reshape(NW, NCHUNK, CHUNK)
  dst_p = jnp.concatenate(
      [dst, jnp.full((EPAD - E,), N, jnp.int32)]).reshape(NW, NCHUNK, CHUNK)

  zerosH = jnp.zeros((RPT, H), jnp.float32)
  onesH = jnp.ones((CHUNK, H), jnp.float32)

  degp = _sc_degree(dst_p, onesH, zerosH)
  dp0 = degp[0]
  dp1 = degp[1]

  g1, dis = _tc_first(x, W1, dp0, dp1)

  b1r = b1.reshape(1, H)
  b2r = b2.reshape(1, H)
  b3r = b3.reshape(1, H)

  a1 = _sc_aggregate(g1, src_p, dst_p, zerosH)
  g2 = _tc_combine(a1[0, :N], a1[1, :N], g1, dis, b1r, W2)
  a2 = _sc_aggregate(g2, src_p, dst_p, zerosH)
  g3 = _tc_combine(a2[0, :N], a2[1, :N], g2, dis, b2r, W3)
  a3 = _sc_aggregate(g3, src_p, dst_p, zerosH)

  batch2d = batch.astype(jnp.int32).reshape(N, 1)
  gfp = _pad2(global_features, G, H)
  m1a = _pad2(m1w[:H], H, H)
  m1g = _pad2(m1w[H:], H, H)
  b1p = _pad2(m1b.reshape(1, -1), 1, H)
  m2p = _pad2(m2w, H, H)
  b2p = _pad2(m2b.reshape(1, -1), 1, H)
  m3p = _pad2(m3w, H, H)
  b3p = _pad2(m3b.reshape(1, -1), 1, H)

  z = _tc_pool_mlp(a3[0, :N], a3[1, :N], g3, dis, b3r, batch2d, gfp,
                   m1a, m1g, b1p, m2p, b2p, m3p, b3p)
  return z[:, :1]


# speedup vs baseline: 2.5472x; 2.5454x over previous
import functools

import jax
import jax.numpy as jnp
from jax import lax
from jax.experimental import pallas as pl
from jax.experimental.pallas import tpu as pltpu
from jax.experimental.pallas import tpu_sc as plsc

N = 10000
E = 320000
G = 64
H = 128

NC = 2
NS = 16
NW = NC * NS
CHUNK = 128
EPW = -(-(E // NW) // CHUNK) * CHUNK
NCHUNK = EPW // CHUNK
EPAD = EPW * NW
ACC_ROWS = EPW
RPT = ACC_ROWS // NS

BLK = 2000
NBLK = N // BLK

_sc_mesh = functools.partial(
    plsc.VectorSubcoreMesh, core_axis_name="c", subcore_axis_name="s",
    num_cores=NC, num_subcores=NS)


def _sc_degree_body(dst_hbm, ones_hbm, zeros_hbm, out_hbm, acc, didx, ones_v):
  c = lax.axis_index("c")
  s = lax.axis_index("s")
  wid = s * NC + c
  pltpu.sync_copy(dst_hbm.at[wid], didx)
  pltpu.sync_copy(ones_hbm, ones_v)
  pltpu.sync_copy(zeros_hbm, acc.at[pl.ds(s * RPT, RPT)])
  plsc.subcore_barrier()

  def body(j, _):
    pltpu.sync_copy(ones_v, acc.at[didx.at[j]], add=True)
    return ()

  lax.fori_loop(0, NCHUNK, body, ())
  plsc.subcore_barrier()
  pltpu.sync_copy(acc.at[pl.ds(s * RPT, RPT)],
                  out_hbm.at[c, pl.ds(s * RPT, RPT)])


@functools.cache
def _make_sc_degree():
  return pl.kernel(
      _sc_degree_body,
      out_type=jax.ShapeDtypeStruct((NC, ACC_ROWS, H), jnp.float32),
      mesh=_sc_mesh(),
      scratch_types=[
          pltpu.VMEM_SHARED((ACC_ROWS, H), jnp.float32),
          pltpu.VMEM((NCHUNK, CHUNK), jnp.int32),
          pltpu.VMEM((CHUNK, H), jnp.float32),
      ],
  )


def _sc_degree(*args):
  return _make_sc_degree()(*args)


NRING = 2
PH = 2
CPP = NCHUNK // PH


def _sc_aggregate_body(g_hbm, src_hbm, dst_hbm, zeros_hbm, out_hbm,
                       acc, sidx, didx, rows):
  c = lax.axis_index("c")
  s = lax.axis_index("s")
  wid = s * NC + c
  pltpu.sync_copy(src_hbm.at[wid], sidx)
  pltpu.sync_copy(dst_hbm.at[wid], didx)
  pltpu.sync_copy(zeros_hbm, acc.at[pl.ds(s * RPT, RPT)])
  plsc.subcore_barrier()

  def body(j, _):
    pltpu.sync_copy(g_hbm.at[sidx.at[j]], rows)
    pltpu.sync_copy(rows, acc.at[didx.at[j]], add=True)
    return ()

  lax.fori_loop(0, NCHUNK, body, ())
  plsc.subcore_barrier()
  pltpu.sync_copy(acc.at[pl.ds(s * RPT, RPT)],
                  out_hbm.at[c, pl.ds(s * RPT, RPT)])


@functools.cache
def _make_sc_aggregate():
  return pl.kernel(
      _sc_aggregate_body,
      out_type=jax.ShapeDtypeStruct((NC, ACC_ROWS, H), jnp.float32),
      mesh=_sc_mesh(),
      scratch_types=[
          pltpu.VMEM_SHARED((ACC_ROWS, H), jnp.float32),
          pltpu.VMEM((NCHUNK, CHUNK), jnp.int32),
          pltpu.VMEM((NCHUNK, CHUNK), jnp.int32),
          pltpu.VMEM((CHUNK, H), jnp.float32),
      ],
  )


def _sc_aggregate(*args):
  return _make_sc_aggregate()(*args)


def _tc_first_body(x_ref, w_ref, dp0_ref, dp1_ref, g_ref, dis_ref):
  deg = dp0_ref[:, 0:1] + dp1_ref[:, 0:1] + 1.0
  dis = lax.rsqrt(deg)
  a = jnp.dot(x_ref[...], w_ref[...], preferred_element_type=jnp.float32)
  g_ref[...] = a * dis
  dis_ref[...] = jnp.broadcast_to(dis, (BLK, H))


def _tc_first(x, w, dp0, dp1):
  return pl.pallas_call(
      _tc_first_body,
      grid=(NBLK,),
      in_specs=[
          pl.BlockSpec((BLK, H), lambda i: (i, 0)),
          pl.BlockSpec((H, H), lambda i: (0, 0)),
          pl.BlockSpec((BLK, H), lambda i: (i, 0)),
          pl.BlockSpec((BLK, H), lambda i: (i, 0)),
      ],
      out_specs=[
          pl.BlockSpec((BLK, H), lambda i: (i, 0)),
          pl.BlockSpec((BLK, H), lambda i: (i, 0)),
      ],
      out_shape=[
          jax.ShapeDtypeStruct((N, H), jnp.float32),
          jax.ShapeDtypeStruct((N, H), jnp.float32),
      ],
  )(x, w, dp0, dp1)


def _tc_combine_body(p0_ref, p1_ref, g_ref, dis_ref, b_ref, w_ref, out_ref):
  h = (p0_ref[...] + p1_ref[...] + g_ref[...]) * dis_ref[...] + b_ref[...]
  h = jnp.maximum(h, 0.0)
  a = jnp.dot(h, w_ref[...], preferred_element_type=jnp.float32)
  out_ref[...] = a * dis_ref[...]


def _tc_combine(p0, p1, g, dis, b, w):
  return pl.pallas_call(
      _tc_combine_body,
      grid=(NBLK,),
      in_specs=[
          pl.BlockSpec((BLK, H), lambda i: (i, 0)),
          pl.BlockSpec((BLK, H), lambda i: (i, 0)),
          pl.BlockSpec((BLK, H), lambda i: (i, 0)),
          pl.BlockSpec((BLK, H), lambda i: (i, 0)),
          pl.BlockSpec((1, H), lambda i: (0, 0)),
          pl.BlockSpec((H, H), lambda i: (0, 0)),
      ],
      out_specs=pl.BlockSpec((BLK, H), lambda i: (i, 0)),
      out_shape=jax.ShapeDtypeStruct((N, H), jnp.float32),
  )(p0, p1, g, dis, b, w)


def _tc_pool_mlp_body(p0_ref, p1_ref, g_ref, dis_ref, b_ref, batch_ref,
                      gf_ref, m1a_ref, m1g_ref, b1_ref, m2_ref, b2_ref,
                      m3_ref, b3_ref, out_ref, sums, cnt):
  i = pl.program_id(0)

  @pl.when(i == 0)
  def _():
    sums[...] = jnp.zeros((G, H), jnp.float32)
    cnt[...] = jnp.zeros((G, H), jnp.float32)

  h = (p0_ref[...] + p1_ref[...] + g_ref[...]) * dis_ref[...] + b_ref[...]
  h = jnp.maximum(h, 0.0)
  ids = lax.broadcasted_iota(jnp.int32, (BLK, G), 1)
  oh = (batch_ref[...] == ids).astype(jnp.float32)
  dn = (((0,), (0,)), ((), ()))
  sums[...] += lax.dot_general(oh, h, dn, preferred_element_type=jnp.float32)
  cnt[...] += lax.dot_general(oh, jnp.ones((BLK, H), jnp.float32), dn,
                              preferred_element_type=jnp.float32)

  @pl.when(i == NBLK - 1)
  def _():
    pooled = sums[...] / jnp.maximum(cnt[...], 1.0)
    z = jnp.dot(pooled, m1a_ref[...], preferred_element_type=jnp.float32)
    z += jnp.dot(gf_ref[...], m1g_ref[...], preferred_element_type=jnp.float32)
    z = jnp.maximum(z + b1_ref[...], 0.0)
    z = jnp.dot(z, m2_ref[...], preferred_element_type=jnp.float32)
    z = jnp.maximum(z + b2_ref[...], 0.0)
    z = jnp.dot(z, m3_ref[...], preferred_element_type=jnp.float32)
    out_ref[...] = z + b3_ref[...]


def _tc_pool_mlp(p0, p1, g, dis, b, batch2d, gfp, m1a, m1g, b1p, m2p, b2p,
                 m3p, b3p):
  full = lambda r, c: pl.BlockSpec((r, c), lambda i: (0, 0))
  return pl.pallas_call(
      _tc_pool_mlp_body,
      grid=(NBLK,),
      in_specs=[
          pl.BlockSpec((BLK, H), lambda i: (i, 0)),
          pl.BlockSpec((BLK, H), lambda i: (i, 0)),
          pl.BlockSpec((BLK, H), lambda i: (i, 0)),
          pl.BlockSpec((BLK, H), lambda i: (i, 0)),
          full(1, H),
          pl.BlockSpec((BLK, 1), lambda i: (i, 0)),
          full(G, H), full(H, H), full(H, H), full(1, H),
          full(H, H), full(1, H), full(H, H), full(1, H),
      ],
      out_specs=pl.BlockSpec((G, H), lambda i: (0, 0)),
      out_shape=jax.ShapeDtypeStruct((G, H), jnp.float32),
      scratch_shapes=[
          pltpu.VMEM((G, H), jnp.float32),
          pltpu.VMEM((G, H), jnp.float32),
      ],
  )(p0, p1, g, dis, b, batch2d, gfp, m1a, m1g, b1p, m2p, b2p, m3p, b3p)


def _pad2(a, rows, cols):
  return jnp.zeros((rows, cols), jnp.float32).at[:a.shape[0], :a.shape[1]].set(a)


def kernel(x, edge_index, batch, global_features, W1, b1, W2, b2, W3, b3,
           m1w, m1b, m2w, m2b, m3w, m3b):
  src = edge_index[0].astype(jnp.int32)
  dst = edge_index[1].astype(jnp.int32)
  pad = jnp.arange(EPAD - E, dtype=jnp.int32)
  src_p = jnp.concatenate(
      [src, pad * 37 % N]).reshape(NW, NCHUNK, CHUNK)
  dst_p = jnp.concatenate(
      [dst, N + pad % (ACC_ROWS - N)]).reshape(NW, NCHUNK, CHUNK)

  zerosH = jnp.zeros((RPT, H), jnp.float32)
  onesH = jnp.ones((CHUNK, H), jnp.float32)

  degp = _sc_degree(dst_p, onesH, zerosH)
  dp0 = degp[0]
  dp1 = degp[1]

  g1, dis = _tc_first(x, W1, dp0, dp1)

  b1r = b1.reshape(1, H)
  b2r = b2.reshape(1, H)
  b3r = b3.reshape(1, H)

  a1 = _sc_aggregate(g1, src_p, dst_p, zerosH)
  g2 = _tc_combine(a1[0, :N], a1[1, :N], g1, dis, b1r, W2)
  a2 = _sc_aggregate(g2, src_p, dst_p, zerosH)
  g3 = _tc_combine(a2[0, :N], a2[1, :N], g2, dis, b2r, W3)
  a3 = _sc_aggregate(g3, src_p, dst_p, zerosH)

  batch2d = batch.astype(jnp.int32).reshape(N, 1)
  gfp = _pad2(global_features, G, H)
  m1a = _pad2(m1w[:H], H, H)
  m1g = _pad2(m1w[H:], H, H)
  b1p = _pad2(m1b.reshape(1, -1), 1, H)
  m2p = _pad2(m2w, H, H)
  b2p = _pad2(m2b.reshape(1, -1), 1, H)
  m3p = _pad2(m3w, H, H)
  b3p = _pad2(m3b.reshape(1, -1), 1, H)

  z = _tc_pool_mlp(a3[0, :N], a3[1, :N], g3, dis, b3r, batch2d, gfp,
                   m1a, m1g, b1p, m2p, b2p, m3p, b3p)
  return z[:, :1]
